# baseline (speedup 1.0000x reference)
import functools

import jax
import jax.numpy as jnp
from jax import lax
from jax.experimental import pallas as pl
from jax.experimental.pallas import tpu as pltpu
from jax.experimental.pallas import tpu_sc as plsc

_LOWBITS = 0x7FF


def _mlp_body(qf_ref, b_ref, w1_ref, b1_ref, w2_ref, b2_ref, out_ref):
    xp = jnp.dot(qf_ref[...], b_ref[...], preferred_element_type=jnp.float32)
    ff = jnp.concatenate([jnp.sin(xp), jnp.cos(xp)], axis=1)
    h = jnp.dot(ff, w1_ref[...], preferred_element_type=jnp.float32) + b1_ref[...]
    h = jnp.maximum(h, 0.0)
    qv = jnp.dot(h, w2_ref[...], preferred_element_type=jnp.float32) + b2_ref[...]
    qn = jnp.maximum(jnp.sqrt(jnp.sum(qv * qv, axis=1, keepdims=True)), 1e-8)
    qhat = qv / qn
    n = qhat.shape[0]
    out_ref[...] = jnp.concatenate(
        [qhat,
         jnp.full((n, 1), 2.0, jnp.float32),
         jnp.zeros((n, 7), jnp.float32)],
        axis=1,
    ).astype(jnp.bfloat16)


def _run_mlp(qf, b_mat, w1, b1, w2, b2):
    n = qf.shape[0]
    return pl.pallas_call(
        _mlp_body,
        out_shape=jax.ShapeDtypeStruct((n, 72), jnp.bfloat16),
    )(qf, b_mat, w1, b1.reshape(1, -1), w2, b2.reshape(1, -1))


def _scan_body(qhat_ref, bank_ref, idx_ref, mask_ref, best_pack, best_idx):
    i = pl.program_id(0)
    blk = bank_ref.shape[0]
    b = bank_ref[...]
    rn = jnp.maximum(jnp.sqrt(jnp.sum(b * b, axis=1, keepdims=True)), 1e-8)
    bhat = b / rn
    bhat_e = jnp.concatenate(
        [bhat,
         jnp.ones((blk, 1), jnp.float32),
         jnp.zeros((blk, 7), jnp.float32)],
        axis=1,
    ).astype(jnp.bfloat16)
    sim2 = lax.dot_general(qhat_ref[...], bhat_e,
                           (((1,), (1,)), ((), ())),
                           preferred_element_type=jnp.float32)
    bits = lax.bitcast_convert_type(sim2, jnp.int32)
    inv = (blk - 1) - lax.broadcasted_iota(jnp.int32, (1, blk), 1)
    packed = (bits & jnp.int32(~_LOWBITS)) | inv
    rowpack = jnp.max(packed, axis=1, keepdims=True)
    col = (blk - 1) - (rowpack & _LOWBITS)
    gidx = i * blk + col

    @pl.when(i == 0)
    def _():
        best_pack[...] = rowpack
        best_idx[...] = gidx

    @pl.when(i > 0)
    def _():
        upd = rowpack > best_pack[...]
        best_pack[...] = jnp.where(upd, rowpack, best_pack[...])
        best_idx[...] = jnp.where(upd, gidx, best_idx[...])

    @pl.when(i == pl.num_programs(0) - 1)
    def _():
        val = lax.bitcast_convert_type(
            best_pack[...] & jnp.int32(~_LOWBITS), jnp.float32) - 2.0
        idx_ref[...] = best_idx[...]
        mask_ref[...] = jnp.broadcast_to(
            jnp.where(val < 0.7, 0.0, 1.0), mask_ref.shape)


def _run_scan(qhat_e, bank, blk):
    n = qhat_e.shape[0]
    m, d = bank.shape
    steps = m // blk
    return pl.pallas_call(
        _scan_body,
        grid=(steps,),
        in_specs=[
            pl.BlockSpec((n, 72), lambda i: (0, 0)),
            pl.BlockSpec((blk, d), lambda i: (i, 0)),
        ],
        out_specs=[
            pl.BlockSpec((n, 1), lambda i: (0, 0)),
            pl.BlockSpec((n, d), lambda i: (0, 0)),
        ],
        out_shape=[
            jax.ShapeDtypeStruct((n, 1), jnp.int32),
            jax.ShapeDtypeStruct((n, d), jnp.float32),
        ],
        scratch_shapes=[
            pltpu.VMEM((n, 1), jnp.int32),
            pltpu.VMEM((n, 1), jnp.int32),
        ],
    )(qhat_e, bank)


@functools.lru_cache(maxsize=None)
def _make_gather(m, d, n):
    info = plsc.get_sparse_core_info()
    nw = info.num_cores * info.num_subcores
    lanes = info.num_lanes
    bpw = n // nw
    mesh = plsc.VectorSubcoreMesh(core_axis_name="c", subcore_axis_name="s")

    @functools.partial(
        pl.kernel,
        mesh=mesh,
        compiler_params=pltpu.CompilerParams(use_tc_tiling_on_sc=False),
        out_type=jax.ShapeDtypeStruct((n, d), jnp.float32),
        scratch_types=[
            pltpu.VMEM((bpw,), jnp.int32),
            pltpu.VMEM((bpw, d), jnp.float32),
            pltpu.VMEM((bpw, d), jnp.float32),
            pltpu.SemaphoreType.DMA,
        ],
    )
    def gather(table_hbm, idx_hbm, maskf_hbm, out_hbm, idx_v, mask_v, rows_v, sem):
        wid = lax.axis_index("s") * info.num_cores + lax.axis_index("c")
        base = wid * bpw
        pltpu.sync_copy(idx_hbm.at[pl.ds(base, bpw)], idx_v)
        cp = pltpu.async_copy(table_hbm.at[idx_v], rows_v, sem)
        pltpu.sync_copy(maskf_hbm.at[pl.ds(base, bpw)], mask_v)
        cp.wait()
        for r in range(bpw):
            for c in range(d // lanes):
                sl = (r, pl.ds(c * lanes, lanes))
                rows_v[sl] = rows_v[sl] * mask_v[sl]
        pltpu.sync_copy(rows_v, out_hbm.at[pl.ds(base, bpw)])

    return gather


def kernel(query_features, memory_bank, B_mat, W1, b1, W2, b2):
    m, d = memory_bank.shape
    n = query_features.shape[0]
    qhat_e = _run_mlp(query_features, B_mat, W1, b1, W2, b2)
    idx, maskf = _run_scan(qhat_e, memory_bank, 1000)
    out = _make_gather(m, d, n)(memory_bank, idx.reshape(-1), maskf)
    return out


# speedup vs baseline: 1.5395x; 1.5395x over previous
import functools

import jax
import jax.numpy as jnp
from jax import lax
from jax.experimental import pallas as pl
from jax.experimental.pallas import tpu as pltpu
from jax.experimental.pallas import tpu_sc as plsc

def _mlp_body(qf_ref, b_ref, w1_ref, b1_ref, w2_ref, b2_ref, out_ref):
    xp = jnp.dot(qf_ref[...], b_ref[...], preferred_element_type=jnp.float32)
    ff = jnp.concatenate([jnp.sin(xp), jnp.cos(xp)], axis=1)
    h = jnp.dot(ff, w1_ref[...], preferred_element_type=jnp.float32) + b1_ref[...]
    h = jnp.maximum(h, 0.0)
    qv = jnp.dot(h, w2_ref[...], preferred_element_type=jnp.float32) + b2_ref[...]
    qn = jnp.maximum(jnp.sqrt(jnp.sum(qv * qv, axis=1, keepdims=True)), 1e-8)
    qhat = qv / qn
    n = qhat.shape[0]
    out_ref[...] = jnp.concatenate(
        [qhat,
         jnp.full((n, 1), 2.0, jnp.float32),
         jnp.zeros((n, 7), jnp.float32)],
        axis=1,
    ).astype(jnp.bfloat16)


def _run_mlp(qf, b_mat, w1, b1, w2, b2):
    n = qf.shape[0]
    return pl.pallas_call(
        _mlp_body,
        out_shape=jax.ShapeDtypeStruct((n, 72), jnp.bfloat16),
    )(qf, b_mat, w1, b1.reshape(1, -1), w2, b2.reshape(1, -1))


_LOWBITS = 0xFFF


def _pack_max(sim2, inv):
    bits = lax.bitcast_convert_type(sim2, jnp.int32)
    packed = (bits & jnp.int32(~_LOWBITS)) | inv
    packed_f = lax.bitcast_convert_type(packed, jnp.float32)
    return jnp.max(packed_f, axis=1, keepdims=True)


def _scan_body(qhat_ref, bank_ref, idx_ref, idx2_ref, mask_ref,
               best_pack, best_idx):
    i = pl.program_id(0)
    blk2, d2 = bank_ref.shape
    d = d2 // 2
    blk = 2 * blk2
    b = bank_ref[...]

    def half(lo):
        bh = b[:, lo:lo + d]
        rinv = jnp.minimum(
            lax.rsqrt(jnp.sum(bh * bh, axis=1, keepdims=True)), 1e8)
        return jnp.concatenate(
            [bh * rinv,
             jnp.ones((blk2, 1), jnp.float32),
             jnp.zeros((blk2, 7), jnp.float32)],
            axis=1,
        ).astype(jnp.bfloat16)

    q = qhat_ref[...]
    nt = (((1,), (1,)), ((), ()))
    sim2_e = lax.dot_general(q, half(0), nt, preferred_element_type=jnp.float32)
    sim2_o = lax.dot_general(q, half(d), nt, preferred_element_type=jnp.float32)
    iota2 = 2 * lax.broadcasted_iota(jnp.int32, (1, blk2), 1)
    rp_e = _pack_max(sim2_e, (blk - 1) - iota2)
    rp_o = _pack_max(sim2_o, (blk - 2) - iota2)
    rowpack = lax.bitcast_convert_type(
        jnp.maximum(rp_e, rp_o), jnp.int32)
    col = (blk - 1) - (rowpack & _LOWBITS)
    gidx = i * blk + col

    @pl.when(i == 0)
    def _():
        best_pack[...] = rowpack
        best_idx[...] = gidx

    @pl.when(i > 0)
    def _():
        upd = rowpack > best_pack[...]
        best_pack[...] = jnp.where(upd, rowpack, best_pack[...])
        best_idx[...] = jnp.where(upd, gidx, best_idx[...])

    @pl.when(i == pl.num_programs(0) - 1)
    def _():
        val = lax.bitcast_convert_type(
            best_pack[...] & jnp.int32(~_LOWBITS), jnp.float32) - 2.0
        idx_ref[...] = best_idx[...]
        idx2_ref[...] = best_idx[...] >> 1
        mask_ref[...] = jnp.where(val < 0.7, 0.0, 1.0)


def _run_scan(qhat_e, bank_pairs, blk):
    n = qhat_e.shape[0]
    m2, d2 = bank_pairs.shape
    steps = (2 * m2) // blk
    return pl.pallas_call(
        _scan_body,
        grid=(steps,),
        in_specs=[
            pl.BlockSpec((n, 72), lambda i: (0, 0)),
            pl.BlockSpec((blk // 2, d2), lambda i: (i, 0)),
        ],
        out_specs=[
            pl.BlockSpec((n, 1), lambda i: (0, 0)),
            pl.BlockSpec((n, 1), lambda i: (0, 0)),
            pl.BlockSpec((n, 1), lambda i: (0, 0)),
        ],
        out_shape=[
            jax.ShapeDtypeStruct((n, 1), jnp.int32),
            jax.ShapeDtypeStruct((n, 1), jnp.int32),
            jax.ShapeDtypeStruct((n, 1), jnp.float32),
        ],
        scratch_shapes=[
            pltpu.VMEM((n, 1), jnp.int32),
            pltpu.VMEM((n, 1), jnp.int32),
        ],
    )(qhat_e, bank_pairs)


@functools.lru_cache(maxsize=None)
def _make_gather(m2, d, n):
    info = plsc.get_sparse_core_info()
    nw = info.num_cores * info.num_subcores
    lanes = info.num_lanes
    bpw = n // nw
    mesh = plsc.VectorSubcoreMesh(core_axis_name="c", subcore_axis_name="s")

    @functools.partial(
        pl.kernel,
        mesh=mesh,
        out_type=jax.ShapeDtypeStruct((n, 2 * d), jnp.float32),
        scratch_types=[
            pltpu.VMEM((bpw,), jnp.int32),
            pltpu.VMEM((bpw, 2 * d), jnp.float32),
            pltpu.SemaphoreType.DMA,
        ],
    )
    def gather(table_hbm, idx2_hbm, out_hbm, idx2_v, pair_v, sem):
        wid = lax.axis_index("s") * info.num_cores + lax.axis_index("c")
        base = wid * bpw
        pltpu.sync_copy(idx2_hbm.at[pl.ds(base, bpw)], idx2_v)
        pltpu.async_copy(table_hbm.at[idx2_v], pair_v, sem).wait()
        pltpu.sync_copy(pair_v, out_hbm.at[pl.ds(base, bpw)])

    return gather


def _epilogue_body(pairs_ref, idx_ref, mask_ref, out_ref):
    d = out_ref.shape[1]
    par = (idx_ref[...] & 1) == 1
    half = jnp.where(par, pairs_ref[:, d:2 * d], pairs_ref[:, 0:d])
    out_ref[...] = half * mask_ref[...]


def _run_epilogue(pairs, idx, mask):
    n, d2 = pairs.shape
    return pl.pallas_call(
        _epilogue_body,
        out_shape=jax.ShapeDtypeStruct((n, d2 // 2), jnp.float32),
    )(pairs, idx, mask)


def kernel(query_features, memory_bank, B_mat, W1, b1, W2, b2):
    m, d = memory_bank.shape
    n = query_features.shape[0]
    qhat_e = _run_mlp(query_features, B_mat, W1, b1, W2, b2)
    bank_pairs = memory_bank.reshape(m // 2, 2 * d)
    idx, idx2, maskf = _run_scan(qhat_e, bank_pairs, 4000)
    pairs = _make_gather(m // 2, d, n)(bank_pairs, idx2.reshape(-1))
    return _run_epilogue(pairs, idx, maskf)
